# Initial kernel scaffold; baseline (speedup 1.0000x reference)
import jax
import jax.numpy as jnp
from jax.experimental import pallas as pl


def kernel(x, edge_index, W1, b1, W2, b2):
    raise NotImplementedError("write your pallas kernel here")


# speedup vs baseline: 11.8880x; 11.8880x over previous
import functools

import jax
import jax.numpy as jnp
from jax import lax
from jax.experimental import pallas as pl
from jax.experimental.pallas import tpu as pltpu
from jax.experimental.pallas import tpu_sc as plsc

_N = 10000
_NP = 10240
_E = 320000
_NW = 32
_ETILE = 10240
_NCHUNK = 80
_EP = _NW * _ETILE
_ROWS_PER_TILE = _NP // 16

@functools.cache
def _get_mesh():
    return plsc.VectorSubcoreMesh(core_axis_name="c", subcore_axis_name="s")


def _deg_body(col_hbm, out_hbm, idx_v, ones_v, dv, sh_deg):
    cid = lax.axis_index("c")
    sid = lax.axis_index("s")
    gid = cid * 16 + sid
    zero16 = jnp.zeros((16,), jnp.float32)
    one16 = jnp.ones((16,), jnp.float32)

    def _z(i, _):
        dv[pl.ds(i * 16, 16)] = zero16
        return 0

    lax.fori_loop(0, _ROWS_PER_TILE // 16, _z, 0)

    def _o(i, _):
        ones_v[pl.ds(i * 16, 16)] = one16
        return 0

    lax.fori_loop(0, 8, _o, 0)

    pltpu.sync_copy(dv, sh_deg.at[pl.ds(sid * _ROWS_PER_TILE, _ROWS_PER_TILE)])
    pltpu.sync_copy(col_hbm.at[gid], idx_v)
    plsc.subcore_barrier()

    def _acc(j, _):
        pltpu.sync_copy(ones_v, sh_deg.at[idx_v.at[j]], add=True)
        return 0

    lax.fori_loop(0, _NCHUNK, _acc, 0)
    plsc.subcore_barrier()
    pltpu.sync_copy(sh_deg.at[pl.ds(sid * _ROWS_PER_TILE, _ROWS_PER_TILE)], dv)
    pltpu.sync_copy(dv, out_hbm.at[cid, pl.ds(sid * _ROWS_PER_TILE, _ROWS_PER_TILE)])


@functools.cache
def _deg_kernel():
    return pl.kernel(
        _deg_body,
        out_type=jax.ShapeDtypeStruct((2, _NP), jnp.float32),
        mesh=_get_mesh(),
        scratch_types=[
            pltpu.VMEM((_NCHUNK, 128), jnp.int32),
            pltpu.VMEM((128,), jnp.float32),
            pltpu.VMEM((_ROWS_PER_TILE,), jnp.float32),
            pltpu.VMEM_SHARED((_NP,), jnp.float32),
        ],
    )


@functools.cache
def _make_scatter(D):
    def _scatter(s_hbm, row_hbm, col_hbm, out_hbm, ir_v, ic_v, rows_v, sem, acc):
        cid = lax.axis_index("c")
        sid = lax.axis_index("s")
        gid = cid * 16 + sid
        zero16 = jnp.zeros((16,), jnp.float32)

        def _z(i, _):
            for t in range(D // 16):
                rows_v[i, pl.ds(t * 16, 16)] = zero16
            return 0

        lax.fori_loop(0, 128, _z, 0)

        def _zc(i, _):
            pltpu.sync_copy(
                rows_v, acc.at[pl.ds(sid * _ROWS_PER_TILE + i * 128, 128)]
            )
            return 0

        lax.fori_loop(0, _ROWS_PER_TILE // 128, _zc, 0)

        pltpu.sync_copy(row_hbm.at[gid], ir_v)
        pltpu.sync_copy(col_hbm.at[gid], ic_v)
        plsc.subcore_barrier()

        def _acc_body(j, _):
            pltpu.async_copy(s_hbm.at[ir_v.at[j]], rows_v, sem).wait()
            pltpu.sync_copy(rows_v, acc.at[ic_v.at[j]], add=True)
            return 0

        lax.fori_loop(0, _NCHUNK, _acc_body, 0)
        plsc.subcore_barrier()
        pltpu.sync_copy(
            acc.at[pl.ds(sid * _ROWS_PER_TILE, _ROWS_PER_TILE)],
            out_hbm.at[cid, pl.ds(sid * _ROWS_PER_TILE, _ROWS_PER_TILE)],
        )

    return pl.kernel(
        _scatter,
        out_type=jax.ShapeDtypeStruct((2, _NP, D), jnp.float32),
        mesh=_get_mesh(),
        compiler_params=pltpu.CompilerParams(use_tc_tiling_on_sc=(D == 128)),
        scratch_types=[
            pltpu.VMEM((_NCHUNK, 128), jnp.int32),
            pltpu.VMEM((_NCHUNK, 128), jnp.int32),
            pltpu.VMEM((128, D), jnp.float32),
            pltpu.SemaphoreType.DMA,
            pltpu.VMEM_SHARED((_NP, D), jnp.float32),
        ],
    )


_BLK = 1024


def _row_mask(shape):
    rid = pl.program_id(0) * _BLK + lax.broadcasted_iota(jnp.int32, shape, 0)
    return rid < _N


def _s1_body(x_ref, d0_ref, d1_ref, w_ref, s_ref, u_ref):
    deg = d0_ref[...] + d1_ref[...] + 1.0
    u = lax.rsqrt(deg)
    h = jnp.dot(x_ref[...], w_ref[...], preferred_element_type=jnp.float32)
    s_ref[...] = jnp.where(_row_mask((_BLK, 1)), u * h, 0.0)
    u_ref[...] = u


def _s3_body(p0_ref, p1_ref, s1_ref, u_ref, w_ref, b_ref, s2_ref):
    u = u_ref[...]
    t = u * (p0_ref[...] + p1_ref[...] + s1_ref[...]) + b_ref[...]
    r = jnp.maximum(t, 0.0)
    h2 = jnp.dot(r, w_ref[...], preferred_element_type=jnp.float32)
    s2_ref[...] = jnp.where(_row_mask((_BLK, 1)), u * h2, 0.0)


def _s5_body(q0_ref, q1_ref, s2_ref, u_ref, b_ref, o_ref):
    o = u_ref[...] * (q0_ref[...] + q1_ref[...] + s2_ref[...]) + b_ref[...]
    m = jnp.max(o, axis=1, keepdims=True)
    lg = o - m
    o_ref[...] = lg - jnp.log(jnp.sum(jnp.exp(lg), axis=1, keepdims=True))


def _node_spec(d):
    return pl.BlockSpec((_BLK, d), lambda i: (i, 0))


def _full_spec(r, c):
    return pl.BlockSpec((r, c), lambda i: (0, 0))


_s1_call = pl.pallas_call(
    _s1_body,
    grid=(10,),
    in_specs=[_node_spec(128), _node_spec(1), _node_spec(1), _full_spec(128, 128)],
    out_specs=[_node_spec(128), _node_spec(1)],
    out_shape=[
        jax.ShapeDtypeStruct((_NP, 128), jnp.float32),
        jax.ShapeDtypeStruct((_NP, 1), jnp.float32),
    ],
)

_s3_call = pl.pallas_call(
    _s3_body,
    grid=(10,),
    in_specs=[
        _node_spec(128),
        _node_spec(128),
        _node_spec(128),
        _node_spec(1),
        _full_spec(128, 32),
        _full_spec(1, 128),
    ],
    out_specs=_node_spec(32),
    out_shape=jax.ShapeDtypeStruct((_NP, 32), jnp.float32),
)

_s5_call = pl.pallas_call(
    _s5_body,
    grid=(10,),
    in_specs=[
        _node_spec(32),
        _node_spec(32),
        _node_spec(32),
        _node_spec(1),
        _full_spec(1, 32),
    ],
    out_specs=_node_spec(32),
    out_shape=jax.ShapeDtypeStruct((_NP, 32), jnp.float32),
)


def kernel(x, edge_index, W1, b1, W2, b2):
    x_p = jnp.zeros((_NP, 128), jnp.float32).at[:_N].set(x)
    pad = jnp.full((_EP - _E,), _N, jnp.int32)
    row_r = jnp.concatenate([edge_index[0], pad]).reshape(_NW, _NCHUNK, 128)
    col_r = jnp.concatenate([edge_index[1], pad]).reshape(_NW, _NCHUNK, 128)

    degp = _deg_kernel()(col_r)
    d0 = degp[0].reshape(_NP, 1)
    d1 = degp[1].reshape(_NP, 1)

    s1, u = _s1_call(x_p, d0, d1, W1.T)
    p = _make_scatter(128)(s1, row_r, col_r)
    s2 = _s3_call(p[0], p[1], s1, u, W2.T, b1.reshape(1, 128))
    q = _make_scatter(32)(s2, row_r, col_r)
    o = _s5_call(q[0], q[1], s2, u, b2.reshape(1, 32))
    return o[:_N]


# speedup vs baseline: 17.7651x; 1.4944x over previous
import functools

import jax
import jax.numpy as jnp
from jax import lax
from jax.experimental import pallas as pl
from jax.experimental.pallas import tpu as pltpu
from jax.experimental.pallas import tpu_sc as plsc

_N = 10000
_NP = 10240
_E = 320000
_NW = 32
_ETILE = 10240
_NCHUNK = 80
_EP = _NW * _ETILE
_ROWS_PER_TILE = _NP // 16

@functools.cache
def _get_mesh():
    return plsc.VectorSubcoreMesh(core_axis_name="c", subcore_axis_name="s")


def _deg_body(col_hbm, out_hbm, idx_v, ones_v, dv, sh_deg):
    cid = lax.axis_index("c")
    sid = lax.axis_index("s")
    gid = cid * 16 + sid
    zero16 = jnp.zeros((16,), jnp.float32)
    one16 = jnp.ones((16,), jnp.float32)

    def _z(i, _):
        dv[pl.ds(i * 16, 16)] = zero16
        return 0

    lax.fori_loop(0, _ROWS_PER_TILE // 16, _z, 0)

    def _o(i, _):
        ones_v[pl.ds(i * 16, 16)] = one16
        return 0

    lax.fori_loop(0, 8, _o, 0)

    pltpu.sync_copy(dv, sh_deg.at[pl.ds(sid * _ROWS_PER_TILE, _ROWS_PER_TILE)])
    pltpu.sync_copy(col_hbm.at[gid], idx_v)
    plsc.subcore_barrier()

    def _acc(j, _):
        pltpu.sync_copy(ones_v, sh_deg.at[idx_v.at[j]], add=True)
        return 0

    lax.fori_loop(0, _NCHUNK, _acc, 0)
    plsc.subcore_barrier()
    pltpu.sync_copy(sh_deg.at[pl.ds(sid * _ROWS_PER_TILE, _ROWS_PER_TILE)], dv)
    pltpu.sync_copy(dv, out_hbm.at[cid, pl.ds(sid * _ROWS_PER_TILE, _ROWS_PER_TILE)])


@functools.cache
def _deg_kernel():
    return pl.kernel(
        _deg_body,
        out_type=jax.ShapeDtypeStruct((2, _NP), jnp.float32),
        mesh=_get_mesh(),
        scratch_types=[
            pltpu.VMEM((_NCHUNK, 128), jnp.int32),
            pltpu.VMEM((128,), jnp.float32),
            pltpu.VMEM((_ROWS_PER_TILE,), jnp.float32),
            pltpu.VMEM_SHARED((_NP,), jnp.float32),
        ],
    )


_NBUF = 4
_NCHUNK2 = 2 * _NCHUNK


@functools.cache
def _make_scatter(D):
    Dh = D // 2

    def _scatter(s_hbm, row_hbm, col_hbm, out_hbm, ir_v, ic_v, rows, sems, acc):
        cid = lax.axis_index("c")
        sid = lax.axis_index("s")
        zero16 = jnp.zeros((16,), jnp.float32)
        off16 = jnp.full((16,), cid * _NP, jnp.int32)

        def _z(i, _):
            for t in range(Dh // 16):
                rows[0][i, pl.ds(t * 16, 16)] = zero16
            return 0

        lax.fori_loop(0, 128, _z, 0)

        def _zc(i, _):
            pltpu.sync_copy(
                rows[0], acc.at[pl.ds(sid * _ROWS_PER_TILE + i * 128, 128)]
            )
            return 0

        lax.fori_loop(0, _ROWS_PER_TILE // 128, _zc, 0)

        pltpu.sync_copy(row_hbm.at[sid], ir_v)
        pltpu.sync_copy(col_hbm.at[sid], ic_v)

        def _ofs(i, _):
            for t in range(8):
                ir_v[i, pl.ds(t * 16, 16)] = ir_v[i, pl.ds(t * 16, 16)] + off16
            return 0

        lax.fori_loop(0, _NCHUNK2, _ofs, 0)
        plsc.subcore_barrier()

        for b in range(_NBUF):
            pltpu.async_copy(s_hbm.at[ir_v.at[b]], rows[b], sems[b])

        def _acc_body(i, _):
            for b in range(_NBUF):
                j = _NBUF * i + b
                pltpu.make_async_copy(s_hbm.at[ir_v.at[j]], rows[b], sems[b]).wait()
                pltpu.sync_copy(rows[b], acc.at[ic_v.at[j]], add=True)

                @pl.when(i < _NCHUNK2 // _NBUF - 1)
                def _():
                    pltpu.async_copy(s_hbm.at[ir_v.at[j + _NBUF]], rows[b], sems[b])

            return 0

        lax.fori_loop(0, _NCHUNK2 // _NBUF, _acc_body, 0)
        plsc.subcore_barrier()
        pltpu.sync_copy(
            acc.at[pl.ds(sid * _ROWS_PER_TILE, _ROWS_PER_TILE)],
            out_hbm.at[cid, pl.ds(sid * _ROWS_PER_TILE, _ROWS_PER_TILE)],
        )

    return pl.kernel(
        _scatter,
        out_type=jax.ShapeDtypeStruct((2, _NP, Dh), jnp.float32),
        mesh=_get_mesh(),
        compiler_params=pltpu.CompilerParams(use_tc_tiling_on_sc=False),
        scratch_types=[
            pltpu.VMEM((_NCHUNK2, 128), jnp.int32),
            pltpu.VMEM((_NCHUNK2, 128), jnp.int32),
            [pltpu.VMEM((128, Dh), jnp.float32) for _ in range(_NBUF)],
            [pltpu.SemaphoreType.DMA for _ in range(_NBUF)],
            pltpu.VMEM_SHARED((_NP, Dh), jnp.float32),
        ],
    )


_BLK = 1024


def _row_mask(shape):
    rid = pl.program_id(0) * _BLK + lax.broadcasted_iota(jnp.int32, shape, 0)
    return rid < _N


def _s1_body(x_ref, d0_ref, d1_ref, w_ref, s_ref, u_ref):
    deg = d0_ref[...] + d1_ref[...] + 1.0
    u = lax.rsqrt(deg)
    h = jnp.dot(x_ref[...], w_ref[...], preferred_element_type=jnp.float32)
    s_ref[...] = jnp.where(_row_mask((_BLK, 1)), u * h, 0.0)
    u_ref[...] = u


def _s3_body(p0_ref, p1_ref, s1_ref, u_ref, w_ref, b_ref, s2_ref):
    u = u_ref[...]
    p = jnp.concatenate([p0_ref[...], p1_ref[...]], axis=1)
    t = u * (p + s1_ref[...]) + b_ref[...]
    r = jnp.maximum(t, 0.0)
    h2 = jnp.dot(r, w_ref[...], preferred_element_type=jnp.float32)
    s2_ref[...] = jnp.where(_row_mask((_BLK, 1)), u * h2, 0.0)


def _s5_body(q0_ref, q1_ref, s2_ref, u_ref, b_ref, o_ref):
    q = jnp.concatenate([q0_ref[...], q1_ref[...]], axis=1)
    o = u_ref[...] * (q + s2_ref[...]) + b_ref[...]
    m = jnp.max(o, axis=1, keepdims=True)
    lg = o - m
    o_ref[...] = lg - jnp.log(jnp.sum(jnp.exp(lg), axis=1, keepdims=True))


def _node_spec(d):
    return pl.BlockSpec((_BLK, d), lambda i: (i, 0))


def _full_spec(r, c):
    return pl.BlockSpec((r, c), lambda i: (0, 0))


_s1_call = pl.pallas_call(
    _s1_body,
    grid=(10,),
    in_specs=[_node_spec(128), _node_spec(1), _node_spec(1), _full_spec(128, 128)],
    out_specs=[_node_spec(128), _node_spec(1)],
    out_shape=[
        jax.ShapeDtypeStruct((_NP, 128), jnp.float32),
        jax.ShapeDtypeStruct((_NP, 1), jnp.float32),
    ],
)

_s3_call = pl.pallas_call(
    _s3_body,
    grid=(10,),
    in_specs=[
        _node_spec(64),
        _node_spec(64),
        _node_spec(128),
        _node_spec(1),
        _full_spec(128, 32),
        _full_spec(1, 128),
    ],
    out_specs=_node_spec(32),
    out_shape=jax.ShapeDtypeStruct((_NP, 32), jnp.float32),
)

_s5_call = pl.pallas_call(
    _s5_body,
    grid=(10,),
    in_specs=[
        _node_spec(16),
        _node_spec(16),
        _node_spec(32),
        _node_spec(1),
        _full_spec(1, 32),
    ],
    out_specs=_node_spec(32),
    out_shape=jax.ShapeDtypeStruct((_NP, 32), jnp.float32),
)


def kernel(x, edge_index, W1, b1, W2, b2):
    x_p = jnp.zeros((_NP, 128), jnp.float32).at[:_N].set(x)
    pad = jnp.full((_EP - _E,), _N, jnp.int32)
    row_p = jnp.concatenate([edge_index[0], pad])
    col_p = jnp.concatenate([edge_index[1], pad])
    col_deg = col_p.reshape(_NW, _NCHUNK, 128)
    row_r = row_p.reshape(16, _NCHUNK2, 128)
    col_r = col_p.reshape(16, _NCHUNK2, 128)

    degp = _deg_kernel()(col_deg)
    d0 = degp[0].reshape(_NP, 1)
    d1 = degp[1].reshape(_NP, 1)

    s1, u = _s1_call(x_p, d0, d1, W1.T)
    s1h = jnp.concatenate([s1[:, :64], s1[:, 64:]], axis=0)
    p = _make_scatter(128)(s1h, row_r, col_r)
    s2 = _s3_call(p[0], p[1], s1, u, W2.T, b1.reshape(1, 128))
    s2h = jnp.concatenate([s2[:, :16], s2[:, 16:]], axis=0)
    q = _make_scatter(32)(s2h, row_r, col_r)
    o = _s5_call(q[0], q[1], s2, u, b2.reshape(1, 32))
    return o[:_N]


# speedup vs baseline: 18.1655x; 1.0225x over previous
import functools

import jax
import jax.numpy as jnp
from jax import lax
from jax.experimental import pallas as pl
from jax.experimental.pallas import tpu as pltpu
from jax.experimental.pallas import tpu_sc as plsc

_N = 10000
_NP = 10240
_E = 320000
_NW = 32
_ETILE = 10240
_NCHUNK = 80
_EP = _NW * _ETILE
_ROWS_PER_TILE = _NP // 16

@functools.cache
def _get_mesh():
    return plsc.VectorSubcoreMesh(core_axis_name="c", subcore_axis_name="s")


def _deg_body(col_hbm, out_hbm, idx_v, ones_v, dv, sh_deg):
    cid = lax.axis_index("c")
    sid = lax.axis_index("s")
    gid = cid * 16 + sid
    zero16 = jnp.zeros((16,), jnp.float32)
    one16 = jnp.ones((16,), jnp.float32)

    def _z(i, _):
        dv[pl.ds(i * 16, 16)] = zero16
        return 0

    lax.fori_loop(0, _ROWS_PER_TILE // 16, _z, 0)

    def _o(i, _):
        ones_v[pl.ds(i * 16, 16)] = one16
        return 0

    lax.fori_loop(0, 8, _o, 0)

    pltpu.sync_copy(dv, sh_deg.at[pl.ds(sid * _ROWS_PER_TILE, _ROWS_PER_TILE)])
    pltpu.sync_copy(col_hbm.at[gid], idx_v)
    plsc.subcore_barrier()

    def _acc(j, _):
        pltpu.sync_copy(ones_v, sh_deg.at[idx_v.at[j]], add=True)
        return 0

    lax.fori_loop(0, _NCHUNK, _acc, 0)
    plsc.subcore_barrier()
    pltpu.sync_copy(sh_deg.at[pl.ds(sid * _ROWS_PER_TILE, _ROWS_PER_TILE)], dv)
    pltpu.sync_copy(dv, out_hbm.at[cid, pl.ds(sid * _ROWS_PER_TILE, _ROWS_PER_TILE)])


@functools.cache
def _deg_kernel():
    return pl.kernel(
        _deg_body,
        out_type=jax.ShapeDtypeStruct((2, _NP), jnp.float32),
        mesh=_get_mesh(),
        scratch_types=[
            pltpu.VMEM((_NCHUNK, 128), jnp.int32),
            pltpu.VMEM((128,), jnp.float32),
            pltpu.VMEM((_ROWS_PER_TILE,), jnp.float32),
            pltpu.VMEM_SHARED((_NP,), jnp.float32),
        ],
    )


_NBUF = 6
_NCHUNK2 = 2 * _NCHUNK


@functools.cache
def _make_scatter(D):
    Dh = D // 2

    def _scatter(s_hbm, row_hbm, col_hbm, out_hbm, ir_v, ic_v, rows, gsems, ssems, acc):
        cid = lax.axis_index("c")
        sid = lax.axis_index("s")
        zero16 = jnp.zeros((16,), jnp.float32)
        off16 = jnp.full((16,), cid * _NP, jnp.int32)

        def _z(i, _):
            for t in range(Dh // 16):
                rows[0][i, pl.ds(t * 16, 16)] = zero16
            return 0

        lax.fori_loop(0, 128, _z, 0)

        def _zc(i, _):
            pltpu.sync_copy(
                rows[0], acc.at[pl.ds(sid * _ROWS_PER_TILE + i * 128, 128)]
            )
            return 0

        lax.fori_loop(0, _ROWS_PER_TILE // 128, _zc, 0)

        pltpu.sync_copy(row_hbm.at[sid], ir_v)
        pltpu.sync_copy(col_hbm.at[sid], ic_v)

        def _ofs(i, _):
            for t in range(8):
                ir_v[i, pl.ds(t * 16, 16)] = ir_v[i, pl.ds(t * 16, 16)] + off16
            return 0

        lax.fori_loop(0, _NCHUNK2, _ofs, 0)
        plsc.subcore_barrier()

        for b in range(_NBUF):
            pltpu.async_copy(s_hbm.at[ir_v.at[b]], rows[b], gsems[b])

        nblk = _NCHUNK2 // _NBUF

        def _acc_body(i, _):
            for b in range(_NBUF):
                j = _NBUF * i + b
                pltpu.make_async_copy(s_hbm.at[ir_v.at[j]], rows[b], gsems[b]).wait()
                pltpu.async_copy(rows[b], acc.at[ic_v.at[j]], ssems[b], add=True)
            for b in range(_NBUF):
                j = _NBUF * i + b

                @pl.when(i < nblk - 1)
                def _():
                    pltpu.make_async_copy(
                        rows[b], acc.at[ic_v.at[j]], ssems[b]
                    ).wait()
                    pltpu.async_copy(s_hbm.at[ir_v.at[j + _NBUF]], rows[b], gsems[b])

            return 0

        lax.fori_loop(0, nblk, _acc_body, 0)
        for b in range(_NBUF):
            pltpu.make_async_copy(
                rows[b], acc.at[ic_v.at[_NCHUNK2 - _NBUF + b]], ssems[b]
            ).wait()
        plsc.subcore_barrier()
        pltpu.sync_copy(
            acc.at[pl.ds(sid * _ROWS_PER_TILE, _ROWS_PER_TILE)],
            out_hbm.at[cid, pl.ds(sid * _ROWS_PER_TILE, _ROWS_PER_TILE)],
        )

    return pl.kernel(
        _scatter,
        out_type=jax.ShapeDtypeStruct((2, _NP, Dh), jnp.float32),
        mesh=_get_mesh(),
        compiler_params=pltpu.CompilerParams(use_tc_tiling_on_sc=False),
        scratch_types=[
            pltpu.VMEM((_NCHUNK2, 128), jnp.int32),
            pltpu.VMEM((_NCHUNK2, 128), jnp.int32),
            [pltpu.VMEM((128, Dh), jnp.float32) for _ in range(_NBUF)],
            [pltpu.SemaphoreType.DMA for _ in range(_NBUF)],
            [pltpu.SemaphoreType.DMA for _ in range(_NBUF)],
            pltpu.VMEM_SHARED((_NP, Dh), jnp.float32),
        ],
    )


_BLK = 1024


def _row_mask(shape):
    rid = pl.program_id(0) * _BLK + lax.broadcasted_iota(jnp.int32, shape, 0)
    return rid < _N


def _s1_body(x_ref, d0_ref, d1_ref, w_ref, s_ref, u_ref):
    deg = d0_ref[...] + d1_ref[...] + 1.0
    u = lax.rsqrt(deg)
    h = jnp.dot(x_ref[...], w_ref[...], preferred_element_type=jnp.float32)
    s_ref[...] = jnp.where(_row_mask((_BLK, 1)), u * h, 0.0)
    u_ref[...] = u


def _s3_body(p0_ref, p1_ref, s1_ref, u_ref, w_ref, b_ref, s2_ref):
    u = u_ref[...]
    p = jnp.concatenate([p0_ref[...], p1_ref[...]], axis=1)
    t = u * (p + s1_ref[...]) + b_ref[...]
    r = jnp.maximum(t, 0.0)
    h2 = jnp.dot(r, w_ref[...], preferred_element_type=jnp.float32)
    s2_ref[...] = jnp.where(_row_mask((_BLK, 1)), u * h2, 0.0)


def _s5_body(q0_ref, q1_ref, s2_ref, u_ref, b_ref, o_ref):
    q = jnp.concatenate([q0_ref[...], q1_ref[...]], axis=1)
    o = u_ref[...] * (q + s2_ref[...]) + b_ref[...]
    m = jnp.max(o, axis=1, keepdims=True)
    lg = o - m
    o_ref[...] = lg - jnp.log(jnp.sum(jnp.exp(lg), axis=1, keepdims=True))


def _node_spec(d):
    return pl.BlockSpec((_BLK, d), lambda i: (i, 0))


def _full_spec(r, c):
    return pl.BlockSpec((r, c), lambda i: (0, 0))


_s1_call = pl.pallas_call(
    _s1_body,
    grid=(10,),
    in_specs=[_node_spec(128), _node_spec(1), _node_spec(1), _full_spec(128, 128)],
    out_specs=[_node_spec(128), _node_spec(1)],
    out_shape=[
        jax.ShapeDtypeStruct((_NP, 128), jnp.float32),
        jax.ShapeDtypeStruct((_NP, 1), jnp.float32),
    ],
)

_s3_call = pl.pallas_call(
    _s3_body,
    grid=(10,),
    in_specs=[
        _node_spec(64),
        _node_spec(64),
        _node_spec(128),
        _node_spec(1),
        _full_spec(128, 32),
        _full_spec(1, 128),
    ],
    out_specs=_node_spec(32),
    out_shape=jax.ShapeDtypeStruct((_NP, 32), jnp.float32),
)

_s5_call = pl.pallas_call(
    _s5_body,
    grid=(10,),
    in_specs=[
        _node_spec(16),
        _node_spec(16),
        _node_spec(32),
        _node_spec(1),
        _full_spec(1, 32),
    ],
    out_specs=_node_spec(32),
    out_shape=jax.ShapeDtypeStruct((_NP, 32), jnp.float32),
)


def kernel(x, edge_index, W1, b1, W2, b2):
    x_p = jnp.zeros((_NP, 128), jnp.float32).at[:_N].set(x)
    pad = jnp.full((_EP - _E,), _N, jnp.int32)
    row_p = jnp.concatenate([edge_index[0], pad])
    col_p = jnp.concatenate([edge_index[1], pad])
    col_deg = col_p.reshape(_NW, _NCHUNK, 128)
    row_r = row_p.reshape(16, _NCHUNK2, 128)
    col_r = col_p.reshape(16, _NCHUNK2, 128)

    degp = _deg_kernel()(col_deg)
    d0 = degp[0].reshape(_NP, 1)
    d1 = degp[1].reshape(_NP, 1)

    s1, u = _s1_call(x_p, d0, d1, W1.T)
    s1h = jnp.concatenate([s1[:, :64], s1[:, 64:]], axis=0)
    p = _make_scatter(128)(s1h, row_r, col_r)
    s2 = _s3_call(p[0], p[1], s1, u, W2.T, b1.reshape(1, 128))
    s2h = jnp.concatenate([s2[:, :16], s2[:, 16:]], axis=0)
    q = _make_scatter(32)(s2h, row_r, col_r)
    o = _s5_call(q[0], q[1], s2, u, b2.reshape(1, 32))
    return o[:_N]


# speedup vs baseline: 18.6496x; 1.0267x over previous
import functools

import jax
import jax.numpy as jnp
from jax import lax
from jax.experimental import pallas as pl
from jax.experimental.pallas import tpu as pltpu
from jax.experimental.pallas import tpu_sc as plsc

_N = 10000
_NP = 10240
_E = 320000
_NW = 32
_ETILE = 10240
_NCHUNK = 80
_EP = _NW * _ETILE
_ROWS_PER_TILE = _NP // 16

@functools.cache
def _get_mesh():
    return plsc.VectorSubcoreMesh(core_axis_name="c", subcore_axis_name="s")


def _deg_body(col_hbm, out_hbm, idx_v, ones_v, dv, sh_deg):
    cid = lax.axis_index("c")
    sid = lax.axis_index("s")
    gid = cid * 16 + sid
    zero16 = jnp.zeros((16,), jnp.float32)
    one16 = jnp.ones((16,), jnp.float32)

    def _z(i, _):
        dv[pl.ds(i * 16, 16)] = zero16
        return 0

    lax.fori_loop(0, _ROWS_PER_TILE // 16, _z, 0)

    def _o(i, _):
        ones_v[pl.ds(i * 16, 16)] = one16
        return 0

    lax.fori_loop(0, 8, _o, 0)

    pltpu.sync_copy(dv, sh_deg.at[pl.ds(sid * _ROWS_PER_TILE, _ROWS_PER_TILE)])
    pltpu.sync_copy(col_hbm.at[gid], idx_v)
    plsc.subcore_barrier()

    def _acc(j, _):
        pltpu.sync_copy(ones_v, sh_deg.at[idx_v.at[j]], add=True)
        return 0

    lax.fori_loop(0, _NCHUNK, _acc, 0)
    plsc.subcore_barrier()
    pltpu.sync_copy(sh_deg.at[pl.ds(sid * _ROWS_PER_TILE, _ROWS_PER_TILE)], dv)
    pltpu.sync_copy(dv, out_hbm.at[cid, pl.ds(sid * _ROWS_PER_TILE, _ROWS_PER_TILE)])


@functools.cache
def _deg_kernel():
    return pl.kernel(
        _deg_body,
        out_type=jax.ShapeDtypeStruct((2, _NP), jnp.float32),
        mesh=_get_mesh(),
        scratch_types=[
            pltpu.VMEM((_NCHUNK, 128), jnp.int32),
            pltpu.VMEM((128,), jnp.float32),
            pltpu.VMEM((_ROWS_PER_TILE,), jnp.float32),
            pltpu.VMEM_SHARED((_NP,), jnp.float32),
        ],
    )


_NBUF = 6
_NCHUNK2 = 2 * _NCHUNK


@functools.cache
def _make_scatter(D):
    Dh = D // 2

    def _scatter(s_hbm, row_hbm, col_hbm, out_hbm, ir_v, ic_v, rows, gsems, ssems, acc):
        cid = lax.axis_index("c")
        sid = lax.axis_index("s")
        zero16 = jnp.zeros((16,), jnp.float32)
        off16 = jnp.full((16,), cid * _NP, jnp.int32)

        def _z(i, _):
            for t in range(Dh // 16):
                rows[0][i, pl.ds(t * 16, 16)] = zero16
            return 0

        lax.fori_loop(0, 128, _z, 0)

        def _zc(i, _):
            pltpu.sync_copy(
                rows[0], acc.at[pl.ds(sid * _ROWS_PER_TILE + i * 128, 128)]
            )
            return 0

        lax.fori_loop(0, _ROWS_PER_TILE // 128, _zc, 0)

        pltpu.sync_copy(row_hbm.at[sid], ir_v)
        pltpu.sync_copy(col_hbm.at[sid], ic_v)

        def _ofs(i, _):
            for t in range(8):
                ir_v[i, pl.ds(t * 16, 16)] = ir_v[i, pl.ds(t * 16, 16)] + off16
            return 0

        lax.fori_loop(0, _NCHUNK2, _ofs, 0)
        plsc.subcore_barrier()

        for b in range(_NBUF):
            pltpu.async_copy(s_hbm.at[ir_v.at[b]], rows[b], gsems[b])

        nblk = _NCHUNK2 // _NBUF

        def _acc_body(i, _):
            for b in range(_NBUF):
                j = _NBUF * i + b
                pltpu.make_async_copy(s_hbm.at[ir_v.at[j]], rows[b], gsems[b]).wait()
                pltpu.sync_copy(rows[b], acc.at[ic_v.at[j]], add=True)

                @pl.when(i < nblk - 1)
                def _():
                    pltpu.async_copy(s_hbm.at[ir_v.at[j + _NBUF]], rows[b], gsems[b])

            return 0

        lax.fori_loop(0, nblk, _acc_body, 0)
        plsc.subcore_barrier()
        pltpu.sync_copy(
            acc.at[pl.ds(sid * _ROWS_PER_TILE, _ROWS_PER_TILE)],
            out_hbm.at[cid, pl.ds(sid * _ROWS_PER_TILE, _ROWS_PER_TILE)],
        )

    return pl.kernel(
        _scatter,
        out_type=jax.ShapeDtypeStruct((2, _NP, Dh), jnp.float32),
        mesh=_get_mesh(),
        compiler_params=pltpu.CompilerParams(use_tc_tiling_on_sc=False),
        scratch_types=[
            pltpu.VMEM((_NCHUNK2, 128), jnp.int32),
            pltpu.VMEM((_NCHUNK2, 128), jnp.int32),
            [pltpu.VMEM((128, Dh), jnp.float32) for _ in range(_NBUF)],
            [pltpu.SemaphoreType.DMA for _ in range(_NBUF)],
            [pltpu.SemaphoreType.DMA for _ in range(_NBUF)],
            pltpu.VMEM_SHARED((_NP, Dh), jnp.float32),
        ],
    )


_BLK = 1024


def _row_mask(shape):
    rid = pl.program_id(0) * _BLK + lax.broadcasted_iota(jnp.int32, shape, 0)
    return rid < _N


def _s1_body(x_ref, d0_ref, d1_ref, w_ref, s_ref, u_ref):
    deg = d0_ref[...] + d1_ref[...] + 1.0
    u = lax.rsqrt(deg)
    h = jnp.dot(x_ref[...], w_ref[...], preferred_element_type=jnp.float32)
    s_ref[...] = jnp.where(_row_mask((_BLK, 1)), u * h, 0.0)
    u_ref[...] = u


def _s3_body(p0_ref, p1_ref, s1_ref, u_ref, w_ref, b_ref, s2_ref):
    u = u_ref[...]
    p = jnp.concatenate([p0_ref[...], p1_ref[...]], axis=1)
    t = u * (p + s1_ref[...]) + b_ref[...]
    r = jnp.maximum(t, 0.0)
    h2 = jnp.dot(r, w_ref[...], preferred_element_type=jnp.float32)
    s2_ref[...] = jnp.where(_row_mask((_BLK, 1)), u * h2, 0.0)


def _s5_body(q0_ref, q1_ref, s2_ref, u_ref, b_ref, o_ref):
    q = jnp.concatenate([q0_ref[...], q1_ref[...]], axis=1)
    o = u_ref[...] * (q + s2_ref[...]) + b_ref[...]
    m = jnp.max(o, axis=1, keepdims=True)
    lg = o - m
    o_ref[...] = lg - jnp.log(jnp.sum(jnp.exp(lg), axis=1, keepdims=True))


def _node_spec(d):
    return pl.BlockSpec((_BLK, d), lambda i: (i, 0))


def _full_spec(r, c):
    return pl.BlockSpec((r, c), lambda i: (0, 0))


_s1_call = pl.pallas_call(
    _s1_body,
    grid=(10,),
    in_specs=[_node_spec(128), _node_spec(1), _node_spec(1), _full_spec(128, 128)],
    out_specs=[_node_spec(128), _node_spec(1)],
    out_shape=[
        jax.ShapeDtypeStruct((_NP, 128), jnp.float32),
        jax.ShapeDtypeStruct((_NP, 1), jnp.float32),
    ],
)

_s3_call = pl.pallas_call(
    _s3_body,
    grid=(10,),
    in_specs=[
        _node_spec(64),
        _node_spec(64),
        _node_spec(128),
        _node_spec(1),
        _full_spec(128, 32),
        _full_spec(1, 128),
    ],
    out_specs=_node_spec(32),
    out_shape=jax.ShapeDtypeStruct((_NP, 32), jnp.float32),
)

_s5_call = pl.pallas_call(
    _s5_body,
    grid=(10,),
    in_specs=[
        _node_spec(16),
        _node_spec(16),
        _node_spec(32),
        _node_spec(1),
        _full_spec(1, 32),
    ],
    out_specs=_node_spec(32),
    out_shape=jax.ShapeDtypeStruct((_NP, 32), jnp.float32),
)


def kernel(x, edge_index, W1, b1, W2, b2):
    x_p = jnp.zeros((_NP, 128), jnp.float32).at[:_N].set(x)
    pad = jnp.full((_EP - _E,), _N, jnp.int32)
    row_p = jnp.concatenate([edge_index[0], pad])
    col_p = jnp.concatenate([edge_index[1], pad])
    col_deg = col_p.reshape(_NW, _NCHUNK, 128)
    row_r = row_p.reshape(16, _NCHUNK2, 128)
    col_r = col_p.reshape(16, _NCHUNK2, 128)

    degp = _deg_kernel()(col_deg)
    d0 = degp[0].reshape(_NP, 1)
    d1 = degp[1].reshape(_NP, 1)

    s1, u = _s1_call(x_p, d0, d1, W1.T)
    s1h = jnp.concatenate([s1[:, :64], s1[:, 64:]], axis=0)
    p = _make_scatter(128)(s1h, row_r, col_r)
    s2 = _s3_call(p[0], p[1], s1, u, W2.T, b1.reshape(1, 128))
    s2h = jnp.concatenate([s2[:, :16], s2[:, 16:]], axis=0)
    q = _make_scatter(32)(s2h, row_r, col_r)
    o = _s5_call(q[0], q[1], s2, u, b2.reshape(1, 32))
    return o[:_N]
